# baseline (speedup 1.0000x reference)
import functools

import jax
import jax.numpy as jnp
from jax import lax
from jax.experimental import pallas as pl
from jax.experimental.pallas import tpu as pltpu
from jax.experimental.pallas import tpu_sc as plsc

_NC = 2
_NS = 16
_NW = _NC * _NS
_LANES = 16


def _table_body(embt_ref, w2_ref, b2_ref, wout_ref, bout_ref, s_ref):
    v = lax.dot_general(
        wout_ref[...], w2_ref[...], (((1,), (0,)), ((), ())),
        preferred_element_type=jnp.float32,
        precision=lax.Precision.HIGHEST,
    )
    c = lax.dot_general(
        wout_ref[...], b2_ref[...], (((1,), (1,)), ((), ())),
        preferred_element_type=jnp.float32,
        precision=lax.Precision.HIGHEST,
    )
    s = lax.dot_general(
        v, embt_ref[...], (((1,), (0,)), ((), ())),
        preferred_element_type=jnp.float32,
        precision=lax.Precision.HIGHEST,
    )
    s_ref[pl.ds(0, s.shape[1])] = (s + c + bout_ref[...])[0]


def _make_gather(f, b, vpad, rows_per_w, unroll):
    mesh = plsc.VectorSubcoreMesh(core_axis_name="c", subcore_axis_name="s")
    active = f // rows_per_w

    @functools.partial(
        pl.kernel,
        out_type=jax.ShapeDtypeStruct((f * b,), jnp.float32),
        mesh=mesh,
        scratch_types=[
            pltpu.VMEM((1, b), jnp.int32),
            pltpu.VMEM((1, b), jnp.int32),
            pltpu.VMEM((b,), jnp.float32),
            pltpu.VMEM((b,), jnp.float32),
            pltpu.VMEM((vpad,), jnp.float32),
            pltpu.SemaphoreType.DMA,
            pltpu.SemaphoreType.DMA,
            pltpu.SemaphoreType.DMA,
            pltpu.SemaphoreType.DMA,
        ],
        compiler_params=pltpu.CompilerParams(needs_layout_passes=False),
    )
    def gather_kernel(table_hbm, xt_hbm, out_hbm, idx_a, idx_b, out_a, out_b,
                      table_v, sem_ia, sem_ib, sem_oa, sem_ob):
        wid = lax.axis_index("s") * _NC + lax.axis_index("c")
        row0 = wid * rows_per_w
        idx_bufs, out_bufs = (idx_a, idx_b), (out_a, out_b)
        sem_in, sem_out = (sem_ia, sem_ib), (sem_oa, sem_ob)

        @pl.when(wid < active)
        def _():
            pltpu.sync_copy(table_hbm, table_v)
            in_h = [None] * rows_per_w
            out_h = [None] * rows_per_w
            in_h[0] = pltpu.async_copy(
                xt_hbm.at[pl.ds(row0, 1)], idx_bufs[0], sem_in[0])
            for k in range(rows_per_w):
                bf = k % 2
                if k + 1 < rows_per_w:
                    in_h[k + 1] = pltpu.async_copy(
                        xt_hbm.at[pl.ds(row0 + k + 1, 1)],
                        idx_bufs[1 - bf], sem_in[1 - bf])
                in_h[k].wait()
                if k >= 2:
                    out_h[k - 2].wait()
                idx_v, out_v = idx_bufs[bf], out_bufs[bf]

                @plsc.parallel_loop(0, b, step=_LANES, unroll=unroll)
                def _(o):
                    idx = idx_v[0, pl.ds(o, _LANES)]
                    out_v[pl.ds(o, _LANES)] = plsc.load_gather(table_v, [idx])

                out_h[k] = pltpu.async_copy(
                    out_v, out_hbm.at[pl.ds((row0 + k) * b, b)], sem_out[bf])
            out_h[rows_per_w - 2].wait()
            out_h[rows_per_w - 1].wait()

    return gather_kernel


def kernel(x, emb, W2, bias2, Wout, bias_out):
    B, F = x.shape
    V, E = emb.shape
    vpad = ((V + 127) // 128) * 128

    table = pl.pallas_call(
        _table_body,
        out_shape=jax.ShapeDtypeStruct((vpad,), jnp.float32),
    )(emb.T, W2, bias2.reshape(1, -1), Wout, bias_out.reshape(1, 1))

    xt = x.T.astype(jnp.int32)
    rows_per_w = max(1, F // _NW)
    while F % rows_per_w:
        rows_per_w += 1
    out_flat = _make_gather(F, B, vpad, rows_per_w, 2)(table, xt)
    return jnp.transpose(out_flat.reshape(F, 1, B), (2, 0, 1))


# speedup vs baseline: 1.1536x; 1.1536x over previous
import functools

import jax
import jax.numpy as jnp
from jax import lax
from jax.experimental import pallas as pl
from jax.experimental.pallas import tpu as pltpu
from jax.experimental.pallas import tpu_sc as plsc

_NC = 2
_NS = 16
_NW = _NC * _NS
_LANES = 16


def _table_body(embt_ref, w2_ref, b2_ref, wout_ref, bout_ref, s_ref):
    v = lax.dot_general(
        wout_ref[...], w2_ref[...], (((1,), (0,)), ((), ())),
        preferred_element_type=jnp.float32,
        precision=lax.Precision.HIGHEST,
    )
    c = lax.dot_general(
        wout_ref[...], b2_ref[...], (((1,), (1,)), ((), ())),
        preferred_element_type=jnp.float32,
        precision=lax.Precision.HIGHEST,
    )
    s = lax.dot_general(
        v, embt_ref[...], (((1,), (0,)), ((), ())),
        preferred_element_type=jnp.float32,
        precision=lax.Precision.HIGHEST,
    )
    s_ref[pl.ds(0, s.shape[1])] = (s + c + bout_ref[...])[0]


def _make_gather(f, b, vpad, rows_per_w, unroll):
    mesh = plsc.VectorSubcoreMesh(core_axis_name="c", subcore_axis_name="s")
    active = f // rows_per_w

    @functools.partial(
        pl.kernel,
        out_type=jax.ShapeDtypeStruct((f * b,), jnp.float32),
        mesh=mesh,
        scratch_types=[
            pltpu.VMEM((1, b), jnp.int32),
            pltpu.VMEM((1, b), jnp.int32),
            pltpu.VMEM((b,), jnp.float32),
            pltpu.VMEM((b,), jnp.float32),
            pltpu.VMEM((vpad,), jnp.float32),
            pltpu.SemaphoreType.DMA,
            pltpu.SemaphoreType.DMA,
            pltpu.SemaphoreType.DMA,
            pltpu.SemaphoreType.DMA,
        ],
        compiler_params=pltpu.CompilerParams(needs_layout_passes=False),
    )
    def gather_kernel(table_hbm, xt_hbm, out_hbm, idx_a, idx_b, out_a, out_b,
                      table_v, sem_ia, sem_ib, sem_oa, sem_ob):
        wid = lax.axis_index("s") * _NC + lax.axis_index("c")
        row0 = wid * rows_per_w
        idx_bufs, out_bufs = (idx_a, idx_b), (out_a, out_b)
        sem_in, sem_out = (sem_ia, sem_ib), (sem_oa, sem_ob)

        @pl.when(wid < active)
        def _():
            pltpu.sync_copy(table_hbm, table_v)
            in_h = [None] * rows_per_w
            out_h = [None] * rows_per_w
            in_h[0] = pltpu.async_copy(
                xt_hbm.at[pl.ds(row0, 1)], idx_bufs[0], sem_in[0])
            for k in range(rows_per_w):
                bf = k % 2
                if k + 1 < rows_per_w:
                    in_h[k + 1] = pltpu.async_copy(
                        xt_hbm.at[pl.ds(row0 + k + 1, 1)],
                        idx_bufs[1 - bf], sem_in[1 - bf])
                in_h[k].wait()
                if k >= 2:
                    out_h[k - 2].wait()
                idx_v, out_v = idx_bufs[bf], out_bufs[bf]

                @plsc.parallel_loop(0, b, step=_LANES, unroll=unroll)
                def _(o):
                    idx = idx_v[0, pl.ds(o, _LANES)]
                    out_v[pl.ds(o, _LANES)] = plsc.load_gather(table_v, [idx])

                out_h[k] = pltpu.async_copy(
                    out_v, out_hbm.at[pl.ds((row0 + k) * b, b)], sem_out[bf])
            out_h[rows_per_w - 2].wait()
            out_h[rows_per_w - 1].wait()

    return gather_kernel


def kernel(x, emb, W2, bias2, Wout, bias_out):
    B, F = x.shape
    V, E = emb.shape
    vpad = ((V + 127) // 128) * 128

    table = pl.pallas_call(
        _table_body,
        out_shape=jax.ShapeDtypeStruct((vpad,), jnp.float32),
    )(emb.T, W2, bias2.reshape(1, -1), Wout, bias_out.reshape(1, 1))

    xt = x.T.astype(jnp.int32)
    rows_per_w = max(1, F // _NW)
    while F % rows_per_w:
        rows_per_w += 1
    out_flat = _make_gather(F, B, vpad, rows_per_w, 8)(table, xt)
    return jnp.transpose(out_flat.reshape(F, 1, B), (2, 0, 1))
